# baseline (speedup 1.0000x reference)
import functools

import jax
import jax.numpy as jnp
from jax import lax
from jax.experimental import pallas as pl
from jax.experimental.pallas import tpu as pltpu
from jax.experimental.pallas import tpu_sc as plsc

_NW = 32


def _sc_zero(B, P, D):
    SLAB = (P + 1) // _NW
    W = 2 * D
    mesh = plsc.VectorSubcoreMesh(core_axis_name="c", subcore_axis_name="s")

    @functools.partial(
        pl.kernel,
        mesh=mesh,
        out_type=jax.ShapeDtypeStruct((B, P + 1, W), jnp.float32),
        scratch_types=[pltpu.VMEM((SLAB, W), jnp.float32),
                       pltpu.SemaphoreType.DMA],
        compiler_params=pltpu.CompilerParams(use_tc_tiling_on_sc=False),
    )
    def k(out_hbm, buf, sem):
        wid = lax.axis_index("s") * 2 + lax.axis_index("c")
        lo = SLAB * wid

        nchunks = SLAB * W // 16

        def zero_body(i, _):
            r = i // (W // 16)
            c = (i % (W // 16)) * 16
            buf[r, pl.ds(c, 16)] = jnp.zeros((16,), jnp.float32)
            return 0

        lax.fori_loop(0, nchunks, zero_body, 0)

        for g in range(0, B, 16):
            cps = [pltpu.make_async_copy(
                buf, out_hbm.at[b, pl.ds(lo, SLAB), :], sem)
                for b in range(g, g + 16)]
            for cp in cps:
                cp.start()
            for cp in cps:
                cp.wait()

    return k


def kernel(x, cls_embedding, pos_embedding_global, pos_embedding_local):
    B, P, D = x.shape
    return _sc_zero(B, P, D)()


# speedup vs baseline: 1.4732x; 1.4732x over previous
import jax
import jax.numpy as jnp
from jax.experimental import pallas as pl

_NUM_GLOBAL = 576
_NUM_LOCAL = 196
_NBATCH = 2


def _body(x_ref, cls_ref, pos_ref, out_ref):
    for i in range(_NBATCH):
        left = jnp.concatenate([cls_ref[...], x_ref[i]], axis=0)
        out_ref[i] = jnp.concatenate([left, pos_ref[...]], axis=1)


def kernel(x, cls_embedding, pos_embedding_global, pos_embedding_local):
    B, P, D = x.shape
    if P == _NUM_GLOBAL:
        pos = pos_embedding_global
    elif P == _NUM_LOCAL:
        pos = pos_embedding_local
    else:
        raise RuntimeError(f"Num patches {P} not matching")
    E = pos.shape[1]
    nb = _NBATCH if B % _NBATCH == 0 else 1

    out = pl.pallas_call(
        _body if nb == _NBATCH else _body1,
        grid=(B // nb,),
        in_specs=[
            pl.BlockSpec((nb, P, D), lambda b: (b, 0, 0)),
            pl.BlockSpec((1, D), lambda b: (0, 0)),
            pl.BlockSpec((P + 1, E), lambda b: (0, 0)),
        ],
        out_specs=pl.BlockSpec((nb, P + 1, D + E), lambda b: (b, 0, 0)),
        out_shape=jax.ShapeDtypeStruct((B, P + 1, D + E), x.dtype),
    )(x, cls_embedding, pos)
    return out


def _body1(x_ref, cls_ref, pos_ref, out_ref):
    left = jnp.concatenate([cls_ref[...], x_ref[0]], axis=0)
    out_ref[0] = jnp.concatenate([left, pos_ref[...]], axis=1)


# speedup vs baseline: 1.4868x; 1.0092x over previous
import jax
import jax.numpy as jnp
from jax.experimental import pallas as pl

_NUM_GLOBAL = 576
_NUM_LOCAL = 196
_NBATCH = 4


def _body(x_ref, cls_ref, pos_ref, out_ref):
    for i in range(_NBATCH):
        left = jnp.concatenate([cls_ref[...], x_ref[i]], axis=0)
        out_ref[i] = jnp.concatenate([left, pos_ref[...]], axis=1)


def kernel(x, cls_embedding, pos_embedding_global, pos_embedding_local):
    B, P, D = x.shape
    if P == _NUM_GLOBAL:
        pos = pos_embedding_global
    elif P == _NUM_LOCAL:
        pos = pos_embedding_local
    else:
        raise RuntimeError(f"Num patches {P} not matching")
    E = pos.shape[1]
    nb = _NBATCH if B % _NBATCH == 0 else 1

    out = pl.pallas_call(
        _body if nb == _NBATCH else _body1,
        grid=(B // nb,),
        in_specs=[
            pl.BlockSpec((nb, P, D), lambda b: (b, 0, 0)),
            pl.BlockSpec((1, D), lambda b: (0, 0)),
            pl.BlockSpec((P + 1, E), lambda b: (0, 0)),
        ],
        out_specs=pl.BlockSpec((nb, P + 1, D + E), lambda b: (b, 0, 0)),
        out_shape=jax.ShapeDtypeStruct((B, P + 1, D + E), x.dtype),
    )(x, cls_embedding, pos)
    return out


def _body1(x_ref, cls_ref, pos_ref, out_ref):
    left = jnp.concatenate([cls_ref[...], x_ref[0]], axis=0)
    out_ref[0] = jnp.concatenate([left, pos_ref[...]], axis=1)
